# baseline (speedup 1.0000x reference)
import jax
import jax.numpy as jnp
from jax import lax
from jax.experimental import pallas as pl
from jax.experimental.pallas import tpu as pltpu
from jax.experimental.pallas import tpu_sc as plsc

BATCH = 16384
DIM = 32
NC = 2
NS = 16
NW = NC * NS
B_PER_W = BATCH // NW
GROUPS = B_PER_W // 16


def _gmf_body(uidx_hbm, iidx_hbm, user_t, item_t, w_hbm, b_hbm,
              out_hbm,
              uidx_v, iidx_v, u_t, i_t, out_v, w_v, b_v,
              sem_u, sem_i):
    wid = lax.axis_index("s") * NC + lax.axis_index("c")
    base = wid * B_PER_W

    pltpu.sync_copy(uidx_hbm.at[pl.ds(base, B_PER_W)], uidx_v)
    pltpu.sync_copy(iidx_hbm.at[pl.ds(base, B_PER_W)], iidx_v)
    pltpu.sync_copy(w_hbm, w_v)
    pltpu.sync_copy(b_hbm, b_v)

    copies = []
    for d in range(DIM):
        copies.append(pltpu.async_copy(
            user_t.at[d].at[uidx_v], u_t.at[d], sem_u))
        copies.append(pltpu.async_copy(
            item_t.at[d].at[iidx_v], i_t.at[d], sem_i))
    for c in copies:
        c.wait()

    w_lo = w_v[pl.ds(0, 16)]
    w_hi = w_v[pl.ds(16, 16)]
    bias = b_v[...]

    def cgroup(g, carry):
        acc = bias
        for d in range(DIM):
            wd = w_lo[d] if d < 16 else w_hi[d - 16]
            u = u_t[d, pl.ds(g * 16, 16)]
            i = i_t[d, pl.ds(g * 16, 16)]
            acc = acc + u * i * wd
        out_v[pl.ds(g * 16, 16)] = 1.0 / (1.0 + jnp.exp(-acc))
        return carry

    lax.fori_loop(0, GROUPS, cgroup, 0)
    pltpu.sync_copy(out_v, out_hbm.at[pl.ds(base, B_PER_W)])


@jax.jit
def kernel(user_indices, item_indices, user_table, item_table, W, b):
    uidx = user_indices.astype(jnp.int32)
    iidx = item_indices.astype(jnp.int32)
    w32 = W.reshape(DIM).astype(jnp.float32)
    b16 = jnp.broadcast_to(b.astype(jnp.float32), (16,))
    user_t = user_table.T
    item_t = item_table.T

    run = pl.kernel(
        _gmf_body,
        out_type=jax.ShapeDtypeStruct((BATCH,), jnp.float32),
        mesh=plsc.VectorSubcoreMesh(core_axis_name="c", subcore_axis_name="s"),
        compiler_params=pltpu.CompilerParams(
            needs_layout_passes=False, use_tc_tiling_on_sc=False),
        scratch_types=[
            pltpu.VMEM((B_PER_W,), jnp.int32),
            pltpu.VMEM((B_PER_W,), jnp.int32),
            pltpu.VMEM((DIM, B_PER_W), jnp.float32),
            pltpu.VMEM((DIM, B_PER_W), jnp.float32),
            pltpu.VMEM((B_PER_W,), jnp.float32),
            pltpu.VMEM((DIM,), jnp.float32),
            pltpu.VMEM((16,), jnp.float32),
            pltpu.SemaphoreType.DMA,
            pltpu.SemaphoreType.DMA,
        ],
    )
    out = run(uidx, iidx, user_t, item_t, w32, b16)
    return out.reshape(BATCH, 1)


# speedup vs baseline: 21.7432x; 21.7432x over previous
import jax
import jax.numpy as jnp
from jax import lax
from jax.experimental import pallas as pl
from jax.experimental.pallas import tpu as pltpu
from jax.experimental.pallas import tpu_sc as plsc

BATCH = 16384
DIM = 32
LANES = 128
NC = 2
NS = 16
NW = NC * NS
B_PER_W = BATCH // NW
GROUPS = B_PER_W // 16
NBUF = 4


def _gmf_body(uidx_hbm, iidx_hbm, user_t, item_t, w_hbm, b_hbm,
              out_hbm,
              uidx_v, iidx_v, ustage, istage, out_v, w_v, b_v,
              usems, isems):
    wid = lax.axis_index("s") * NC + lax.axis_index("c")
    base = wid * B_PER_W

    pltpu.sync_copy(uidx_hbm.at[pl.ds(base, B_PER_W)], uidx_v)
    pltpu.sync_copy(iidx_hbm.at[pl.ds(base, B_PER_W)], iidx_v)
    pltpu.sync_copy(w_hbm, w_v)
    pltpu.sync_copy(b_hbm, b_v)

    lanes = lax.iota(jnp.int32, 16)
    w_lo = w_v[pl.ds(0, 16)]
    w_hi = w_v[pl.ds(16, 16)]
    bias = b_v[...]

    def fire(slot, ucol, icol):
        ublk = pl.multiple_of((ucol >> 7) * LANES, LANES)
        iblk = pl.multiple_of((icol >> 7) * LANES, LANES)
        pltpu.async_copy(user_t.at[:, pl.ds(ublk, LANES)],
                         ustage.at[slot], usems.at[slot])
        pltpu.async_copy(item_t.at[:, pl.ds(iblk, LANES)],
                         istage.at[slot], isems.at[slot])

    ugrp0 = uidx_v[pl.ds(0, 16)]
    igrp0 = iidx_v[pl.ds(0, 16)]
    for k in range(NBUF):
        fire(k, ugrp0[k], igrp0[k])

    def group(g, carry):
        ugrp = uidx_v[pl.ds(g * 16, 16)]
        igrp = iidx_v[pl.ds(g * 16, 16)]
        nstart = jnp.minimum(g * 16 + 16, B_PER_W - 16)
        ugrp_n = uidx_v[pl.ds(nstart, 16)]
        igrp_n = iidx_v[pl.ds(nstart, 16)]
        acc = bias
        for k in range(16):
            slot = k % NBUF
            pltpu.make_async_copy(user_t.at[:, pl.ds(0, LANES)],
                                  ustage.at[slot], usems.at[slot]).wait()
            pltpu.make_async_copy(item_t.at[:, pl.ds(0, LANES)],
                                  istage.at[slot], isems.at[slot]).wait()
            ul = jnp.full((16,), ugrp[k] & (LANES - 1), jnp.int32)
            il = jnp.full((16,), igrp[k] & (LANES - 1), jnp.int32)
            gu_lo = plsc.load_gather(ustage.at[slot], [lanes, ul])
            gu_hi = plsc.load_gather(ustage.at[slot], [lanes + 16, ul])
            gi_lo = plsc.load_gather(istage.at[slot], [lanes, il])
            gi_hi = plsc.load_gather(istage.at[slot], [lanes + 16, il])
            p = gu_lo * gi_lo * w_lo + gu_hi * gi_hi * w_hi
            s = jnp.sum(p)
            acc = acc + jnp.where(lanes == k, s, 0.0)
            if k + NBUF < 16:
                fire(slot, ugrp[k + NBUF], igrp[k + NBUF])
            else:
                fire(slot, ugrp_n[k + NBUF - 16], igrp_n[k + NBUF - 16])
        out_v[pl.ds(g * 16, 16)] = 1.0 / (1.0 + jnp.exp(-acc))
        return carry

    lax.fori_loop(0, GROUPS, group, 0)

    for k in range(NBUF):
        pltpu.make_async_copy(user_t.at[:, pl.ds(0, LANES)],
                              ustage.at[k], usems.at[k]).wait()
        pltpu.make_async_copy(item_t.at[:, pl.ds(0, LANES)],
                              istage.at[k], isems.at[k]).wait()

    pltpu.sync_copy(out_v, out_hbm.at[pl.ds(base, B_PER_W)])


@jax.jit
def kernel(user_indices, item_indices, user_table, item_table, W, b):
    uidx = user_indices.astype(jnp.int32)
    iidx = item_indices.astype(jnp.int32)
    w32 = W.reshape(DIM).astype(jnp.float32)
    b16 = jnp.broadcast_to(b.astype(jnp.float32), (16,))
    user_t = user_table.T
    item_t = item_table.T

    run = pl.kernel(
        _gmf_body,
        out_type=jax.ShapeDtypeStruct((BATCH,), jnp.float32),
        mesh=plsc.VectorSubcoreMesh(core_axis_name="c", subcore_axis_name="s"),
        compiler_params=pltpu.CompilerParams(
            needs_layout_passes=False, use_tc_tiling_on_sc=True),
        scratch_types=[
            pltpu.VMEM((B_PER_W,), jnp.int32),
            pltpu.VMEM((B_PER_W,), jnp.int32),
            pltpu.VMEM((NBUF, DIM, LANES), jnp.float32),
            pltpu.VMEM((NBUF, DIM, LANES), jnp.float32),
            pltpu.VMEM((B_PER_W,), jnp.float32),
            pltpu.VMEM((DIM,), jnp.float32),
            pltpu.VMEM((16,), jnp.float32),
            pltpu.SemaphoreType.DMA((NBUF,)),
            pltpu.SemaphoreType.DMA((NBUF,)),
        ],
    )
    out = run(uidx, iidx, user_t, item_t, w32, b16)
    return out.reshape(BATCH, 1)


# speedup vs baseline: 21.8755x; 1.0061x over previous
import jax
import jax.numpy as jnp
from jax import lax
from jax.experimental import pallas as pl
from jax.experimental.pallas import tpu as pltpu
from jax.experimental.pallas import tpu_sc as plsc

BATCH = 16384
DIM = 32
LANES = 128
NC = 2
NS = 16
NW = NC * NS
B_PER_W = BATCH // NW
GROUPS = B_PER_W // 16
NBUF = 8


def _gmf_body(uidx_hbm, iidx_hbm, user_t, item_t, w_hbm, b_hbm,
              out_hbm,
              uidx_v, iidx_v, ustage, istage, out_v, w_v, b_v,
              usems, isems):
    wid = lax.axis_index("s") * NC + lax.axis_index("c")
    base = wid * B_PER_W

    pltpu.sync_copy(uidx_hbm.at[pl.ds(base, B_PER_W)], uidx_v)
    pltpu.sync_copy(iidx_hbm.at[pl.ds(base, B_PER_W)], iidx_v)
    pltpu.sync_copy(w_hbm, w_v)
    pltpu.sync_copy(b_hbm, b_v)

    lanes = lax.iota(jnp.int32, 16)
    w_lo = w_v[pl.ds(0, 16)]
    w_hi = w_v[pl.ds(16, 16)]
    bias = b_v[...]

    def fire(slot, ucol, icol):
        ublk = pl.multiple_of((ucol >> 7) * LANES, LANES)
        iblk = pl.multiple_of((icol >> 7) * LANES, LANES)
        pltpu.async_copy(user_t.at[:, pl.ds(ublk, LANES)],
                         ustage.at[slot], usems.at[slot])
        pltpu.async_copy(item_t.at[:, pl.ds(iblk, LANES)],
                         istage.at[slot], isems.at[slot])

    ugrp0 = uidx_v[pl.ds(0, 16)]
    igrp0 = iidx_v[pl.ds(0, 16)]
    for k in range(NBUF):
        fire(k, ugrp0[k], igrp0[k])

    def group(g, carry):
        ugrp = uidx_v[pl.ds(g * 16, 16)]
        igrp = iidx_v[pl.ds(g * 16, 16)]
        nstart = jnp.minimum(g * 16 + 16, B_PER_W - 16)
        ugrp_n = uidx_v[pl.ds(nstart, 16)]
        igrp_n = iidx_v[pl.ds(nstart, 16)]
        acc = bias
        for k in range(16):
            slot = k % NBUF
            pltpu.make_async_copy(user_t.at[:, pl.ds(0, LANES)],
                                  ustage.at[slot], usems.at[slot]).wait()
            pltpu.make_async_copy(item_t.at[:, pl.ds(0, LANES)],
                                  istage.at[slot], isems.at[slot]).wait()
            ul = jnp.full((16,), ugrp[k] & (LANES - 1), jnp.int32)
            il = jnp.full((16,), igrp[k] & (LANES - 1), jnp.int32)
            gu_lo = plsc.load_gather(ustage.at[slot], [lanes, ul])
            gu_hi = plsc.load_gather(ustage.at[slot], [lanes + 16, ul])
            gi_lo = plsc.load_gather(istage.at[slot], [lanes, il])
            gi_hi = plsc.load_gather(istage.at[slot], [lanes + 16, il])
            p = gu_lo * gi_lo * w_lo + gu_hi * gi_hi * w_hi
            s = jnp.sum(p)
            acc = acc + jnp.where(lanes == k, s, 0.0)
            if k + NBUF < 16:
                fire(slot, ugrp[k + NBUF], igrp[k + NBUF])
            else:
                fire(slot, ugrp_n[k + NBUF - 16], igrp_n[k + NBUF - 16])
        out_v[pl.ds(g * 16, 16)] = 1.0 / (1.0 + jnp.exp(-acc))
        return carry

    lax.fori_loop(0, GROUPS, group, 0)

    for k in range(NBUF):
        pltpu.make_async_copy(user_t.at[:, pl.ds(0, LANES)],
                              ustage.at[k], usems.at[k]).wait()
        pltpu.make_async_copy(item_t.at[:, pl.ds(0, LANES)],
                              istage.at[k], isems.at[k]).wait()

    pltpu.sync_copy(out_v, out_hbm.at[pl.ds(base, B_PER_W)])


@jax.jit
def kernel(user_indices, item_indices, user_table, item_table, W, b):
    uidx = user_indices.astype(jnp.int32)
    iidx = item_indices.astype(jnp.int32)
    w32 = W.reshape(DIM).astype(jnp.float32)
    b16 = jnp.broadcast_to(b.astype(jnp.float32), (16,))
    user_t = user_table.T
    item_t = item_table.T

    run = pl.kernel(
        _gmf_body,
        out_type=jax.ShapeDtypeStruct((BATCH,), jnp.float32),
        mesh=plsc.VectorSubcoreMesh(core_axis_name="c", subcore_axis_name="s"),
        compiler_params=pltpu.CompilerParams(
            needs_layout_passes=False, use_tc_tiling_on_sc=True),
        scratch_types=[
            pltpu.VMEM((B_PER_W,), jnp.int32),
            pltpu.VMEM((B_PER_W,), jnp.int32),
            pltpu.VMEM((NBUF, DIM, LANES), jnp.float32),
            pltpu.VMEM((NBUF, DIM, LANES), jnp.float32),
            pltpu.VMEM((B_PER_W,), jnp.float32),
            pltpu.VMEM((DIM,), jnp.float32),
            pltpu.VMEM((16,), jnp.float32),
            pltpu.SemaphoreType.DMA((NBUF,)),
            pltpu.SemaphoreType.DMA((NBUF,)),
        ],
    )
    out = run(uidx, iidx, user_t, item_t, w32, b16)
    return out.reshape(BATCH, 1)
